# Initial kernel scaffold; baseline (speedup 1.0000x reference)
import jax
import jax.numpy as jnp
from jax.experimental import pallas as pl


def kernel(x):
    raise NotImplementedError("write your pallas kernel here")


# speedup vs baseline: 6.2606x; 6.2606x over previous
import functools

import jax
import jax.numpy as jnp
import numpy as np
from jax import lax
from jax.experimental import pallas as pl
from jax.experimental.pallas import tpu as pltpu
from jax.experimental.pallas import tpu_sc as plsc

_NC = 2
_NS = 16
_LANES = 16

_PERM = (2, 0, 4, 5, 7, 6, 3, 1)


def _flip24(bits):
    sgn = jnp.right_shift(bits, 31)
    flipped = jnp.bitwise_xor(bits, jnp.bitwise_or(sgn, jnp.int32(-(2**31))))
    return jnp.bitwise_and(jnp.right_shift(flipped, 8), jnp.int32(0x00FFFFFF))


def _efdmix_build(B, C, N, G):
    S = B * C
    CH_PER_SC = C // _NC
    ROUNDS = CH_PER_SC // 2
    NV = N // _LANES
    GV = G // _LANES
    NW = N // G
    mesh = plsc.VectorSubcoreMesh(
        core_axis_name="c", subcore_axis_name="s", num_cores=_NC,
        num_subcores=_NS,
    )

    @functools.partial(
        pl.kernel,
        out_type=(
            jax.ShapeDtypeStruct((S * N,), jnp.float32),
            jax.ShapeDtypeStruct((S * N,), jnp.float32),
        ),
        mesh=mesh,
        compiler_params=pltpu.CompilerParams(needs_layout_passes=False),
        scratch_types=[
            pltpu.VMEM((N,), jnp.int32),
            pltpu.VMEM((N,), jnp.float32),
            pltpu.VMEM((16 * 256,), jnp.int32),
            pltpu.VMEM((256,), jnp.int32),
            pltpu.VMEM((G,), jnp.int32),
            pltpu.VMEM((G,), jnp.float32),
            pltpu.VMEM((G,), jnp.float32),
            pltpu.VMEM((16,), jnp.float32),
            pltpu.VMEM((16,), jnp.float32),
            pltpu.SemaphoreType.DMA,
        ],
    )
    def efdmix(x_hbm, l0_hbm, l1_hbm, out_hbm, vals_hbm, bufA, bufB, hist16,
               offs, gidx, pwin, xwin, lvec0, lvec1, sem):
        core = lax.axis_index("c")
        sub = lax.axis_index("s")
        b = lax.rem(sub, 8)
        ch_off = lax.div(sub, 8)
        pb = jnp.int32(_PERM[0])
        for kk in range(1, 8):
            pb = jnp.where(b == kk, jnp.int32(_PERM[kk]), pb)

        iota = lax.iota(jnp.int32, _LANES)
        ones_i = jnp.ones((_LANES,), jnp.int32)
        cnt_cal, _ = plsc.scan_count(jnp.zeros((_LANES,), jnp.int32))
        beta_vec = cnt_cal - iota
        gam_vec = plsc.cumsum(ones_i) - iota

        def zero_hist():
            def zb(k2, c_):
                hist16[pl.ds(k2 * 16, 16)] = jnp.zeros((_LANES,), jnp.int32)
                return c_
            lax.fori_loop(0, 256, zb, 0)

        def build_offsets():
            def ob(kv, run):
                tot = jnp.zeros((_LANES,), jnp.int32)
                def lb(ln, t):
                    return t + hist16[pl.ds(ln * 256 + kv * 16, 16)]
                tot = lax.fori_loop(0, 16, lb, tot)
                c = plsc.cumsum(tot)
                exc = c - tot * gam_vec
                offs[pl.ds(kv * 16, 16)] = exc + run
                return run + jnp.sum(tot)
            lax.fori_loop(0, 16, ob, jnp.int32(0))

        def radix_pass(loader, dst_store, payload):
            zero_hist()
            def hb(i, c_):
                d = loader(i)
                plsc.addupdate_scatter(hist16, [iota * 256 + d], ones_i)
                return c_
            lax.fori_loop(0, NV, hb, 0)
            build_offsets()
            def pb_(i, c_):
                d = loader(i)
                w = payload(i)
                cnt, last = plsc.scan_count(d)
                basev = plsc.load_gather(offs, [d])
                dest = basev + cnt - beta_vec
                dst_store(dest, w)
                plsc.store_scatter(offs, [d], dest + 1, mask=last)
                return c_
            lax.fori_loop(0, NV, pb_, 0)

        def round_body(r, carry):
            ch = core * CH_PER_SC + 2 * r + ch_off
            sl = b * C + ch
            base = sl * N
            pbase = (pb * C + ch) * N
            pltpu.sync_copy(x_hbm.at[pl.ds(base, N)], bufB)
            pltpu.sync_copy(l0_hbm.at[pl.ds(sl * 16, 16)], lvec0)
            pltpu.sync_copy(l1_hbm.at[pl.ds(sl * 16, 16)], lvec1)

            def ld0(i):
                v = bufB[pl.ds(i * 16, 16)]
                kb = _flip24(plsc.bitcast(v, jnp.int32))
                return jnp.bitwise_and(kb, jnp.int32(0xFF))
            def pay0(i):
                v = bufB[pl.ds(i * 16, 16)]
                kb = _flip24(plsc.bitcast(v, jnp.int32))
                keyhi = jnp.right_shift(kb, 8)
                return jnp.bitwise_or(jnp.left_shift(keyhi, 16),
                                      i * 16 + iota)
            def st0(dest, w):
                plsc.store_scatter(bufA, [dest], w)
            radix_pass(ld0, st0, pay0)

            def ld1(i):
                w = plsc.bitcast(bufA[pl.ds(i * 16, 16)], jnp.int32)
                return jnp.bitwise_and(jnp.right_shift(w, 16), jnp.int32(0xFF))
            def pay1(i):
                return plsc.bitcast(bufA[pl.ds(i * 16, 16)], jnp.int32)
            def st1(dest, w):
                plsc.store_scatter(bufB, [dest], plsc.bitcast(w, jnp.float32))
            radix_pass(ld1, st1, pay1)

            def ld2(i):
                w = plsc.bitcast(bufB[pl.ds(i * 16, 16)], jnp.int32)
                return jnp.bitwise_and(jnp.right_shift(w, 24), jnp.int32(0xFF))
            def pay2(i):
                return plsc.bitcast(bufB[pl.ds(i * 16, 16)], jnp.int32)
            def st2(dest, w):
                plsc.store_scatter(bufA, [dest], w)
            radix_pass(ld2, st2, pay2)

            def gwin(wi, c_):
                woff = wi * G
                def gx(i, c2_):
                    w = bufA[pl.ds(woff + i * 16, 16)]
                    gidx[pl.ds(i * 16, 16)] = (
                        jnp.bitwise_and(w, jnp.int32(0xFFFF)) + base)
                    return c2_
                lax.fori_loop(0, GV, gx, 0)
                pltpu.async_copy(x_hbm.at[gidx], pwin, sem).wait()
                pltpu.sync_copy(pwin, vals_hbm.at[pl.ds(base + woff, G)])
                return c_
            lax.fori_loop(0, NW, gwin, 0)
            plsc.subcore_barrier()

            l1v = lvec1[...]
            def mwin(wi, c_):
                woff = wi * G
                pltpu.sync_copy(vals_hbm.at[pl.ds(pbase + woff, G)], pwin)
                def mx(i, c2_):
                    w = bufA[pl.ds(woff + i * 16, 16)]
                    idxv = jnp.bitwise_and(w, jnp.int32(0xFFFF))
                    vp = pwin[pl.ds(i * 16, 16)]
                    plsc.store_scatter(bufB, [idxv], vp * l1v)
                    return c2_
                lax.fori_loop(0, GV, mx, 0)
                return c_
            lax.fori_loop(0, NW, mwin, 0)

            l0v = lvec0[...]
            def fwin(wi, c_):
                woff = wi * G
                pltpu.sync_copy(x_hbm.at[pl.ds(base + woff, G)], xwin)
                def fx(i, c2_):
                    j = i * 16
                    xwin[pl.ds(j, 16)] = (bufB[pl.ds(woff + j, 16)]
                                          + l0v * xwin[pl.ds(j, 16)])
                    return c2_
                lax.fori_loop(0, GV, fx, 0)
                pltpu.sync_copy(xwin, out_hbm.at[pl.ds(base + woff, G)])
                return c_
            lax.fori_loop(0, NW, fwin, 0)
            plsc.subcore_barrier()
            return carry

        lax.fori_loop(0, ROUNDS, round_body, 0)

    return efdmix


def kernel(x):
    B, C, W, H = x.shape
    N = W * H
    k = jax.random.key(42)
    k1, _ = jax.random.split(k)
    lmda = jax.random.beta(k1, 0.1, 0.1, (B, 1, 1)).astype(x.dtype).reshape(B)
    l_s = jnp.repeat(lmda, C)
    l0 = jnp.repeat(l_s, 16)
    l1 = jnp.repeat(1.0 - l_s, 16)
    fn = _efdmix_build(B, C, N, G=6272)
    out, _ = fn(x.reshape(B * C * N), l0, l1)
    return out.reshape(B, C, W, H)


# speedup vs baseline: 6.6086x; 1.0556x over previous
import functools

import jax
import jax.numpy as jnp
import numpy as np
from jax import lax
from jax.experimental import pallas as pl
from jax.experimental.pallas import tpu as pltpu
from jax.experimental.pallas import tpu_sc as plsc

_NC = 2
_NS = 16
_LANES = 16

_PERM = (2, 0, 4, 5, 7, 6, 3, 1)


def _flip24(bits):
    sgn = jnp.right_shift(bits, 31)
    flipped = jnp.bitwise_xor(bits, jnp.bitwise_or(sgn, jnp.int32(-(2**31))))
    return jnp.bitwise_and(jnp.right_shift(flipped, 8), jnp.int32(0x00FFFFFF))


def _efdmix_build(B, C, N, G):
    S = B * C
    CH_PER_SC = C // _NC
    ROUNDS = CH_PER_SC // 2
    NV = N // _LANES
    GV = G // _LANES
    NW = N // G
    mesh = plsc.VectorSubcoreMesh(
        core_axis_name="c", subcore_axis_name="s", num_cores=_NC,
        num_subcores=_NS,
    )

    @functools.partial(
        pl.kernel,
        out_type=(
            jax.ShapeDtypeStruct((S * N,), jnp.float32),
            jax.ShapeDtypeStruct((S * N,), jnp.float32),
        ),
        mesh=mesh,
        compiler_params=pltpu.CompilerParams(needs_layout_passes=False),
        scratch_types=[
            pltpu.VMEM((N,), jnp.int32),
            pltpu.VMEM((N,), jnp.float32),
            pltpu.VMEM((16 * 256,), jnp.int32),
            pltpu.VMEM((256,), jnp.int32),
            pltpu.VMEM((G,), jnp.int32),
            pltpu.VMEM((G,), jnp.float32),
            pltpu.VMEM((G,), jnp.float32),
            pltpu.VMEM((16,), jnp.float32),
            pltpu.VMEM((16,), jnp.float32),
            pltpu.SemaphoreType.DMA,
        ],
    )
    def efdmix(x_hbm, l0_hbm, l1_hbm, out_hbm, vals_hbm, bufA, bufB, hist16,
               offs, gidx, pwin, xwin, lvec0, lvec1, sem):
        core = lax.axis_index("c")
        sub = lax.axis_index("s")
        b = lax.rem(sub, 8)
        ch_off = lax.div(sub, 8)
        pb = jnp.int32(_PERM[0])
        for kk in range(1, 8):
            pb = jnp.where(b == kk, jnp.int32(_PERM[kk]), pb)

        iota = lax.iota(jnp.int32, _LANES)
        ones_i = jnp.ones((_LANES,), jnp.int32)
        cnt_cal, _ = plsc.scan_count(jnp.zeros((_LANES,), jnp.int32))
        beta_vec = cnt_cal - iota
        gam_vec = plsc.cumsum(ones_i) - iota

        def unrolled(n, u, body):
            def ub(i, c_):
                for q in range(u):
                    body(i * u + q)
                return c_
            lax.fori_loop(0, n // u, ub, 0)

        def zero_hist():
            def zb(k2):
                hist16[pl.ds(k2 * 16, 16)] = jnp.zeros((_LANES,), jnp.int32)
            unrolled(256, 4, zb)

        def build_offsets():
            def ob(kv, run):
                tot = jnp.zeros((_LANES,), jnp.int32)
                def lb(ln, t):
                    return t + hist16[pl.ds(ln * 256 + kv * 16, 16)]
                tot = lax.fori_loop(0, 16, lb, tot)
                c = plsc.cumsum(tot)
                exc = c - tot * gam_vec
                offs[pl.ds(kv * 16, 16)] = exc + run
                return run + jnp.sum(tot)
            lax.fori_loop(0, 16, ob, jnp.int32(0))

        def radix_pass(loader, dst_store, payload):
            zero_hist()
            def hb(i):
                d = loader(i)
                plsc.addupdate_scatter(hist16, [iota * 256 + d], ones_i)
            unrolled(NV, 4, hb)
            build_offsets()
            def pb_(i):
                d = loader(i)
                w = payload(i)
                cnt, last = plsc.scan_count(d)
                basev = plsc.load_gather(offs, [d])
                dest = basev + cnt - beta_vec
                dst_store(dest, w)
                plsc.store_scatter(offs, [d], dest + 1, mask=last)
            unrolled(NV, 4, pb_)

        def round_body(r, carry):
            ch = core * CH_PER_SC + 2 * r + ch_off
            sl = b * C + ch
            base = sl * N
            pbase = (pb * C + ch) * N
            pltpu.sync_copy(x_hbm.at[pl.ds(base, N)], bufB)
            pltpu.sync_copy(l0_hbm.at[pl.ds(sl * 16, 16)], lvec0)
            pltpu.sync_copy(l1_hbm.at[pl.ds(sl * 16, 16)], lvec1)

            def ld0(i):
                v = bufB[pl.ds(i * 16, 16)]
                kb = _flip24(plsc.bitcast(v, jnp.int32))
                return jnp.bitwise_and(kb, jnp.int32(0xFF))
            def pay0(i):
                v = bufB[pl.ds(i * 16, 16)]
                kb = _flip24(plsc.bitcast(v, jnp.int32))
                keyhi = jnp.right_shift(kb, 8)
                return jnp.bitwise_or(jnp.left_shift(keyhi, 16),
                                      i * 16 + iota)
            def st0(dest, w):
                plsc.store_scatter(bufA, [dest], w)
            radix_pass(ld0, st0, pay0)

            def ld1(i):
                w = plsc.bitcast(bufA[pl.ds(i * 16, 16)], jnp.int32)
                return jnp.bitwise_and(jnp.right_shift(w, 16), jnp.int32(0xFF))
            def pay1(i):
                return plsc.bitcast(bufA[pl.ds(i * 16, 16)], jnp.int32)
            def st1(dest, w):
                plsc.store_scatter(bufB, [dest], plsc.bitcast(w, jnp.float32))
            radix_pass(ld1, st1, pay1)

            def ld2(i):
                w = plsc.bitcast(bufB[pl.ds(i * 16, 16)], jnp.int32)
                return jnp.bitwise_and(jnp.right_shift(w, 24), jnp.int32(0xFF))
            def pay2(i):
                return plsc.bitcast(bufB[pl.ds(i * 16, 16)], jnp.int32)
            def st2(dest, w):
                plsc.store_scatter(bufA, [dest], w)
            radix_pass(ld2, st2, pay2)

            def gwin(wi, c_):
                woff = wi * G
                def gx(i):
                    w = bufA[pl.ds(woff + i * 16, 16)]
                    gidx[pl.ds(i * 16, 16)] = (
                        jnp.bitwise_and(w, jnp.int32(0xFFFF)) + base)
                unrolled(GV, 4, gx)
                pltpu.async_copy(x_hbm.at[gidx], pwin, sem).wait()
                pltpu.sync_copy(pwin, vals_hbm.at[pl.ds(base + woff, G)])
                return c_
            lax.fori_loop(0, NW, gwin, 0)
            plsc.subcore_barrier()

            l1v = lvec1[...]
            def mwin(wi, c_):
                woff = wi * G
                pltpu.sync_copy(vals_hbm.at[pl.ds(pbase + woff, G)], pwin)
                def mx(i):
                    w = bufA[pl.ds(woff + i * 16, 16)]
                    idxv = jnp.bitwise_and(w, jnp.int32(0xFFFF))
                    vp = pwin[pl.ds(i * 16, 16)]
                    plsc.store_scatter(bufB, [idxv], vp * l1v)
                unrolled(GV, 4, mx)
                return c_
            lax.fori_loop(0, NW, mwin, 0)

            l0v = lvec0[...]
            def fwin(wi, c_):
                woff = wi * G
                pltpu.sync_copy(x_hbm.at[pl.ds(base + woff, G)], xwin)
                def fx(i):
                    j = i * 16
                    xwin[pl.ds(j, 16)] = (bufB[pl.ds(woff + j, 16)]
                                          + l0v * xwin[pl.ds(j, 16)])
                unrolled(GV, 4, fx)
                pltpu.sync_copy(xwin, out_hbm.at[pl.ds(base + woff, G)])
                return c_
            lax.fori_loop(0, NW, fwin, 0)
            plsc.subcore_barrier()
            return carry

        lax.fori_loop(0, ROUNDS, round_body, 0)

    return efdmix


def kernel(x):
    B, C, W, H = x.shape
    N = W * H
    k = jax.random.key(42)
    k1, _ = jax.random.split(k)
    lmda = jax.random.beta(k1, 0.1, 0.1, (B, 1, 1)).astype(x.dtype).reshape(B)
    l_s = jnp.repeat(lmda, C)
    l0 = jnp.repeat(l_s, 16)
    l1 = jnp.repeat(1.0 - l_s, 16)
    fn = _efdmix_build(B, C, N, G=6272)
    out, _ = fn(x.reshape(B * C * N), l0, l1)
    return out.reshape(B, C, W, H)


# speedup vs baseline: 7.8966x; 1.1949x over previous
import functools

import jax
import jax.numpy as jnp
import numpy as np
from jax import lax
from jax.experimental import pallas as pl
from jax.experimental.pallas import tpu as pltpu
from jax.experimental.pallas import tpu_sc as plsc

_NC = 2
_NS = 16
_LANES = 16

_PERM = (2, 0, 4, 5, 7, 6, 3, 1)


def _flip24(bits):
    sgn = jnp.right_shift(bits, 31)
    flipped = jnp.bitwise_xor(bits, jnp.bitwise_or(sgn, jnp.int32(-(2**31))))
    return jnp.bitwise_and(jnp.right_shift(flipped, 8), jnp.int32(0x00FFFFFF))


def _efdmix_build(B, C, N, G):
    S = B * C
    CH_PER_SC = C // _NC
    ROUNDS = CH_PER_SC // 2
    NV = N // _LANES
    GV = G // _LANES
    NW = N // G
    mesh = plsc.VectorSubcoreMesh(
        core_axis_name="c", subcore_axis_name="s", num_cores=_NC,
        num_subcores=_NS,
    )

    @functools.partial(
        pl.kernel,
        out_type=(
            jax.ShapeDtypeStruct((S * N,), jnp.float32),
            jax.ShapeDtypeStruct((S * N,), jnp.float32),
        ),
        mesh=mesh,
        compiler_params=pltpu.CompilerParams(needs_layout_passes=False),
        scratch_types=[
            pltpu.VMEM((N,), jnp.int32),
            pltpu.VMEM((N,), jnp.float32),
            pltpu.VMEM((16 * 256,), jnp.int32),
            pltpu.VMEM((256,), jnp.int32),
            pltpu.VMEM((G,), jnp.int32),
            pltpu.VMEM((G,), jnp.float32),
            pltpu.VMEM((G,), jnp.float32),
            pltpu.VMEM((16,), jnp.float32),
            pltpu.VMEM((16,), jnp.float32),
            pltpu.SemaphoreType.DMA,
        ],
    )
    def efdmix(x_hbm, l0_hbm, l1_hbm, out_hbm, vals_hbm, bufA, bufB, hist16,
               offs, gidx, pwin, xwin, lvec0, lvec1, sem):
        core = lax.axis_index("c")
        sub = lax.axis_index("s")
        b = lax.rem(sub, 8)
        ch_off = lax.div(sub, 8)
        pb = jnp.int32(_PERM[0])
        for kk in range(1, 8):
            pb = jnp.where(b == kk, jnp.int32(_PERM[kk]), pb)

        iota = lax.iota(jnp.int32, _LANES)
        ones_i = jnp.ones((_LANES,), jnp.int32)
        cnt_cal, _ = plsc.scan_count(jnp.zeros((_LANES,), jnp.int32))
        beta_vec = cnt_cal - iota
        gam_vec = plsc.cumsum(ones_i) - iota

        def unrolled(n, u, body):
            def ub(i, c_):
                for q in range(u):
                    body(i * u + q)
                return c_
            lax.fori_loop(0, n // u, ub, 0)

        def zero_hist():
            def zb(k2):
                hist16[pl.ds(k2 * 16, 16)] = jnp.zeros((_LANES,), jnp.int32)
            unrolled(256, 4, zb)

        def build_offsets():
            def ob(kv, run):
                tot = jnp.zeros((_LANES,), jnp.int32)
                def lb(ln, t):
                    return t + hist16[pl.ds(ln * 256 + kv * 16, 16)]
                tot = lax.fori_loop(0, 16, lb, tot)
                c = plsc.cumsum(tot)
                exc = c - tot * gam_vec
                offs[pl.ds(kv * 16, 16)] = exc + run
                return run + jnp.sum(tot)
            lax.fori_loop(0, 16, ob, jnp.int32(0))

        def radix_pass(loader, dst_store, payload, pre_hist, next_digit):
            if pre_hist:
                zero_hist()
                def hb(i):
                    d = loader(i)
                    plsc.addupdate_scatter(hist16, [iota * 256 + d], ones_i)
                unrolled(NV, 4, hb)
            build_offsets()
            if next_digit is not None:
                zero_hist()
            def pb_(i):
                d = loader(i)
                w = payload(i)
                cnt, last = plsc.scan_count(d)
                basev = plsc.load_gather(offs, [d])
                dest = basev + cnt - beta_vec
                dst_store(dest, w)
                plsc.store_scatter(offs, [d], dest + 1, mask=last)
                if next_digit is not None:
                    dn = next_digit(w)
                    plsc.addupdate_scatter(hist16, [iota * 256 + dn], ones_i)
            unrolled(NV, 4, pb_)

        def round_body(r, carry):
            ch = core * CH_PER_SC + 2 * r + ch_off
            sl = b * C + ch
            base = sl * N
            pbase = (pb * C + ch) * N
            pltpu.sync_copy(x_hbm.at[pl.ds(base, N)], bufB)
            pltpu.sync_copy(l0_hbm.at[pl.ds(sl * 16, 16)], lvec0)
            pltpu.sync_copy(l1_hbm.at[pl.ds(sl * 16, 16)], lvec1)

            def ld0(i):
                v = bufB[pl.ds(i * 16, 16)]
                kb = _flip24(plsc.bitcast(v, jnp.int32))
                return jnp.bitwise_and(kb, jnp.int32(0xFF))
            def pay0(i):
                v = bufB[pl.ds(i * 16, 16)]
                kb = _flip24(plsc.bitcast(v, jnp.int32))
                keyhi = jnp.right_shift(kb, 8)
                return jnp.bitwise_or(jnp.left_shift(keyhi, 16),
                                      i * 16 + iota)
            def st0(dest, w):
                plsc.store_scatter(bufA, [dest], w)
            def nd1(w):
                return jnp.bitwise_and(jnp.right_shift(w, 16), jnp.int32(0xFF))
            radix_pass(ld0, st0, pay0, pre_hist=True, next_digit=nd1)

            def ld1(i):
                w = plsc.bitcast(bufA[pl.ds(i * 16, 16)], jnp.int32)
                return jnp.bitwise_and(jnp.right_shift(w, 16), jnp.int32(0xFF))
            def pay1(i):
                return plsc.bitcast(bufA[pl.ds(i * 16, 16)], jnp.int32)
            def st1(dest, w):
                plsc.store_scatter(bufB, [dest], plsc.bitcast(w, jnp.float32))
            def nd2(w):
                return jnp.bitwise_and(jnp.right_shift(w, 24), jnp.int32(0xFF))
            radix_pass(ld1, st1, pay1, pre_hist=False, next_digit=nd2)

            def ld2(i):
                w = plsc.bitcast(bufB[pl.ds(i * 16, 16)], jnp.int32)
                return jnp.bitwise_and(jnp.right_shift(w, 24), jnp.int32(0xFF))
            def pay2(i):
                return plsc.bitcast(bufB[pl.ds(i * 16, 16)], jnp.int32)
            def st2(dest, w):
                plsc.store_scatter(bufA, [dest], w)
            radix_pass(ld2, st2, pay2, pre_hist=False, next_digit=None)

            def gwin(wi, c_):
                woff = wi * G
                def gx(i):
                    w = bufA[pl.ds(woff + i * 16, 16)]
                    gidx[pl.ds(i * 16, 16)] = (
                        jnp.bitwise_and(w, jnp.int32(0xFFFF)) + base)
                unrolled(GV, 4, gx)
                pltpu.async_copy(x_hbm.at[gidx], pwin, sem).wait()
                pltpu.sync_copy(pwin, vals_hbm.at[pl.ds(base + woff, G)])
                return c_
            lax.fori_loop(0, NW, gwin, 0)
            plsc.subcore_barrier()

            l0v = lvec0[...]
            l1v = lvec1[...]
            def mwin(wi, c_):
                woff = wi * G
                pltpu.sync_copy(vals_hbm.at[pl.ds(pbase + woff, G)], pwin)
                pltpu.sync_copy(vals_hbm.at[pl.ds(base + woff, G)], xwin)
                def mx(i):
                    w = bufA[pl.ds(woff + i * 16, 16)]
                    idxv = jnp.bitwise_and(w, jnp.int32(0xFFFF))
                    vp = pwin[pl.ds(i * 16, 16)]
                    vo = xwin[pl.ds(i * 16, 16)]
                    plsc.store_scatter(bufB, [idxv], l0v * vo + l1v * vp)
                unrolled(GV, 4, mx)
                return c_
            lax.fori_loop(0, NW, mwin, 0)
            pltpu.sync_copy(bufB, out_hbm.at[pl.ds(base, N)])
            plsc.subcore_barrier()
            return carry

        lax.fori_loop(0, ROUNDS, round_body, 0)

    return efdmix


def kernel(x):
    B, C, W, H = x.shape
    N = W * H
    k = jax.random.key(42)
    k1, _ = jax.random.split(k)
    lmda = jax.random.beta(k1, 0.1, 0.1, (B, 1, 1)).astype(x.dtype).reshape(B)
    l_s = jnp.repeat(lmda, C)
    l0 = jnp.repeat(l_s, 16)
    l1 = jnp.repeat(1.0 - l_s, 16)
    fn = _efdmix_build(B, C, N, G=6272)
    out, _ = fn(x.reshape(B * C * N), l0, l1)
    return out.reshape(B, C, W, H)


# speedup vs baseline: 8.4467x; 1.0697x over previous
import functools

import jax
import jax.numpy as jnp
import numpy as np
from jax import lax
from jax.experimental import pallas as pl
from jax.experimental.pallas import tpu as pltpu
from jax.experimental.pallas import tpu_sc as plsc

_NC = 2
_NS = 16
_LANES = 16

_PERM = (2, 0, 4, 5, 7, 6, 3, 1)


def _flip24(bits):
    sgn = jnp.right_shift(bits, 31)
    flipped = jnp.bitwise_xor(bits, jnp.bitwise_or(sgn, jnp.int32(-(2**31))))
    return jnp.bitwise_and(jnp.right_shift(flipped, 8), jnp.int32(0x00FFFFFF))


def _efdmix_build(B, C, N, G):
    S = B * C
    CH_PER_SC = C // _NC
    ROUNDS = CH_PER_SC // 2
    NV = N // _LANES
    GV = G // _LANES
    NW = N // G
    mesh = plsc.VectorSubcoreMesh(
        core_axis_name="c", subcore_axis_name="s", num_cores=_NC,
        num_subcores=_NS,
    )

    @functools.partial(
        pl.kernel,
        out_type=(
            jax.ShapeDtypeStruct((S * N,), jnp.float32),
            jax.ShapeDtypeStruct((S * N,), jnp.float32),
        ),
        mesh=mesh,
        compiler_params=pltpu.CompilerParams(needs_layout_passes=False),
        scratch_types=[
            pltpu.VMEM((N,), jnp.int32),
            pltpu.VMEM((N,), jnp.float32),
            pltpu.VMEM((16 * 256,), jnp.int32),
            pltpu.VMEM((256,), jnp.int32),
            pltpu.VMEM((G,), jnp.int32),
            pltpu.VMEM((G,), jnp.int32),
            pltpu.VMEM((G,), jnp.float32),
            pltpu.VMEM((G,), jnp.float32),
            pltpu.VMEM((G,), jnp.float32),
            pltpu.VMEM((16,), jnp.float32),
            pltpu.VMEM((16,), jnp.float32),
            pltpu.SemaphoreType.DMA,
            pltpu.SemaphoreType.DMA,
            pltpu.SemaphoreType.DMA,
            pltpu.SemaphoreType.DMA,
        ],
    )
    def efdmix(x_hbm, l0_hbm, l1_hbm, out_hbm, vals_hbm, bufA, bufB, hist16,
               offs, gidxA, gidxB, pwinA, pwinB, xwin, lvec0, lvec1,
               sg0, sg1, sp0, sp1):
        core = lax.axis_index("c")
        sub = lax.axis_index("s")
        b = lax.rem(sub, 8)
        ch_off = lax.div(sub, 8)
        pb = jnp.int32(_PERM[0])
        for kk in range(1, 8):
            pb = jnp.where(b == kk, jnp.int32(_PERM[kk]), pb)

        iota = lax.iota(jnp.int32, _LANES)
        ones_i = jnp.ones((_LANES,), jnp.int32)
        cnt_cal, _ = plsc.scan_count(jnp.zeros((_LANES,), jnp.int32))
        beta_vec = cnt_cal - iota
        gam_vec = plsc.cumsum(ones_i) - iota

        def unrolled(n, u, body):
            def ub(i, c_):
                for q in range(u):
                    body(i * u + q)
                return c_
            lax.fori_loop(0, n // u, ub, 0)

        def zero_hist():
            def zb(k2):
                hist16[pl.ds(k2 * 16, 16)] = jnp.zeros((_LANES,), jnp.int32)
            unrolled(256, 4, zb)

        def build_offsets():
            def ob(kv, run):
                tot = jnp.zeros((_LANES,), jnp.int32)
                def lb(ln, t):
                    return t + hist16[pl.ds(ln * 256 + kv * 16, 16)]
                tot = lax.fori_loop(0, 16, lb, tot)
                c = plsc.cumsum(tot)
                exc = c - tot * gam_vec
                offs[pl.ds(kv * 16, 16)] = exc + run
                return run + jnp.sum(tot)
            lax.fori_loop(0, 16, ob, jnp.int32(0))

        def radix_pass(loader, dst_store, payload, pre_hist, next_digit):
            if pre_hist:
                zero_hist()
                def hb(i):
                    d = loader(i)
                    plsc.addupdate_scatter(hist16, [iota * 256 + d], ones_i)
                unrolled(NV, 4, hb)
            build_offsets()
            if next_digit is not None:
                zero_hist()
            def pb_(i):
                d = loader(i)
                w = payload(i)
                cnt, last = plsc.scan_count(d)
                basev = plsc.load_gather(offs, [d])
                dest = basev + cnt - beta_vec
                dst_store(dest, w)
                plsc.store_scatter(offs, [d], dest + 1, mask=last)
                if next_digit is not None:
                    dn = next_digit(w)
                    plsc.addupdate_scatter(hist16, [iota * 256 + dn], ones_i)
            unrolled(NV, 4, pb_)

        def round_body(r, carry):
            ch = core * CH_PER_SC + 2 * r + ch_off
            sl = b * C + ch
            base = sl * N
            pbase = (pb * C + ch) * N
            pltpu.sync_copy(x_hbm.at[pl.ds(base, N)], bufB)
            pltpu.sync_copy(l0_hbm.at[pl.ds(sl * 16, 16)], lvec0)
            pltpu.sync_copy(l1_hbm.at[pl.ds(sl * 16, 16)], lvec1)

            def ld0(i):
                v = bufB[pl.ds(i * 16, 16)]
                kb = _flip24(plsc.bitcast(v, jnp.int32))
                return jnp.bitwise_and(kb, jnp.int32(0xFF))
            def pay0(i):
                v = bufB[pl.ds(i * 16, 16)]
                kb = _flip24(plsc.bitcast(v, jnp.int32))
                keyhi = jnp.right_shift(kb, 8)
                return jnp.bitwise_or(jnp.left_shift(keyhi, 16),
                                      i * 16 + iota)
            def st0(dest, w):
                plsc.store_scatter(bufA, [dest], w)
            def nd1(w):
                return jnp.bitwise_and(jnp.right_shift(w, 16), jnp.int32(0xFF))
            radix_pass(ld0, st0, pay0, pre_hist=True, next_digit=nd1)

            def ld1(i):
                w = plsc.bitcast(bufA[pl.ds(i * 16, 16)], jnp.int32)
                return jnp.bitwise_and(jnp.right_shift(w, 16), jnp.int32(0xFF))
            def pay1(i):
                return plsc.bitcast(bufA[pl.ds(i * 16, 16)], jnp.int32)
            def st1(dest, w):
                plsc.store_scatter(bufB, [dest], plsc.bitcast(w, jnp.float32))
            def nd2(w):
                return jnp.bitwise_and(jnp.right_shift(w, 24), jnp.int32(0xFF))
            radix_pass(ld1, st1, pay1, pre_hist=False, next_digit=nd2)

            def ld2(i):
                w = plsc.bitcast(bufB[pl.ds(i * 16, 16)], jnp.int32)
                return jnp.bitwise_and(jnp.right_shift(w, 24), jnp.int32(0xFF))
            def pay2(i):
                return plsc.bitcast(bufB[pl.ds(i * 16, 16)], jnp.int32)
            def st2(dest, w):
                plsc.store_scatter(bufA, [dest], w)
            radix_pass(ld2, st2, pay2, pre_hist=False, next_digit=None)

            gidxs = [gidxA, gidxB]
            pwins = [pwinA, pwinB]
            sems_g = [sg0, sg1]
            sems_p = [sp0, sp1]

            def extract(wi, sl_):
                woff = wi * G
                def gx(i):
                    w = bufA[pl.ds(woff + i * 16, 16)]
                    gidxs[sl_][pl.ds(i * 16, 16)] = (
                        jnp.bitwise_and(w, jnp.int32(0xFFFF)) + base)
                unrolled(GV, 4, gx)

            gathers = []
            pubs = [None, None]
            extract(0, 0)
            gathers.append(pltpu.async_copy(
                x_hbm.at[gidxA], pwinA, sems_g[0]))
            for wi in range(1, NW + 1):
                sl_ = wi % 2
                if wi < NW:
                    extract(wi, sl_)
                    if pubs[sl_] is not None:
                        pubs[sl_].wait()
                    gathers.append(pltpu.async_copy(
                        x_hbm.at[gidxs[sl_]], pwins[sl_], sems_g[sl_]))
                prev = (wi - 1) % 2
                gathers[wi - 1].wait()
                pubs[prev] = pltpu.async_copy(
                    pwins[prev],
                    vals_hbm.at[pl.ds(base + (wi - 1) * G, G)], sems_p[prev])
            for p in pubs:
                if p is not None:
                    p.wait()
            plsc.subcore_barrier()

            l0v = lvec0[...]
            l1v = lvec1[...]
            def mwin(wi, c_):
                woff = wi * G
                cp = pltpu.async_copy(
                    vals_hbm.at[pl.ds(pbase + woff, G)], pwinA, sg0)
                co = pltpu.async_copy(
                    vals_hbm.at[pl.ds(base + woff, G)], xwin, sp0)
                cp.wait()
                co.wait()
                def mx(i):
                    w = bufA[pl.ds(woff + i * 16, 16)]
                    idxv = jnp.bitwise_and(w, jnp.int32(0xFFFF))
                    vp = pwinA[pl.ds(i * 16, 16)]
                    vo = xwin[pl.ds(i * 16, 16)]
                    plsc.store_scatter(bufB, [idxv], l0v * vo + l1v * vp)
                unrolled(GV, 4, mx)
                return c_
            lax.fori_loop(0, NW, mwin, 0)
            pltpu.sync_copy(bufB, out_hbm.at[pl.ds(base, N)])
            plsc.subcore_barrier()
            return carry

        lax.fori_loop(0, ROUNDS, round_body, 0)

    return efdmix


def kernel(x):
    B, C, W, H = x.shape
    N = W * H
    k = jax.random.key(42)
    k1, _ = jax.random.split(k)
    lmda = jax.random.beta(k1, 0.1, 0.1, (B, 1, 1)).astype(x.dtype).reshape(B)
    l_s = jnp.repeat(lmda, C)
    l0 = jnp.repeat(l_s, 16)
    l1 = jnp.repeat(1.0 - l_s, 16)
    fn = _efdmix_build(B, C, N, G=3584)
    out, _ = fn(x.reshape(B * C * N), l0, l1)
    return out.reshape(B, C, W, H)
